# baseline (speedup 1.0000x reference)
import functools

import jax
import jax.numpy as jnp
from jax import lax
from jax.experimental import pallas as pl
from jax.experimental.pallas import tpu as pltpu
from jax.experimental.pallas import tpu_sc as plsc

_SC_ROWS = 2048
_CHUNK_ROWS = 32
_NBUF = 2
_TC_BLK = 512


def _sc_copy(table, sc_rows, hidden, dtype):
    info = plsc.get_sparse_core_info()
    num_workers = info.num_cores * info.num_subcores
    rows_per_w = sc_rows // num_workers
    assert rows_per_w * num_workers == sc_rows
    assert rows_per_w % _CHUNK_ROWS == 0
    n_chunks = rows_per_w // _CHUNK_ROWS

    mesh = plsc.VectorSubcoreMesh(core_axis_name="c", subcore_axis_name="s")

    @functools.partial(
        pl.kernel,
        mesh=mesh,
        out_type=jax.ShapeDtypeStruct((sc_rows, hidden), dtype),
        scratch_types=[
            pltpu.VMEM((_NBUF, _CHUNK_ROWS, hidden), dtype),
            pltpu.SemaphoreType.DMA((_NBUF,)),
            pltpu.SemaphoreType.DMA((_NBUF,)),
        ],
    )
    def copy_rows(table_hbm, out_hbm, buf, lsem, ssem):
        wid = lax.axis_index("s") * info.num_cores + lax.axis_index("c")
        base = wid * rows_per_w

        def start_load(g, b):
            return pltpu.async_copy(
                table_hbm.at[pl.ds(base + g * _CHUNK_ROWS, _CHUNK_ROWS)],
                buf.at[b],
                lsem.at[b],
            )

        def start_store(g, b):
            return pltpu.async_copy(
                buf.at[b],
                out_hbm.at[pl.ds(base + g * _CHUNK_ROWS, _CHUNK_ROWS)],
                ssem.at[b],
            )

        loads = [start_load(g, g) for g in range(min(_NBUF, n_chunks))]
        stores = [None] * _NBUF
        for g in range(n_chunks):
            b = g % _NBUF
            loads[b].wait()
            stores[b] = start_store(g, b)
            nxt = g + _NBUF
            if nxt < n_chunks:
                stores[b].wait()
                stores[b] = None
                loads[b] = start_load(nxt, b)
        for h in stores:
            if h is not None:
                h.wait()

    return copy_rows(table)


def kernel(x, table):
    seq_length = x.shape[1]
    num_rows, hidden = table.shape
    seq_length = min(seq_length, num_rows)
    sc_rows = _SC_ROWS
    tc_rows = seq_length - sc_rows
    assert tc_rows % _TC_BLK == 0 and sc_rows % _TC_BLK == 0

    sc_out = _sc_copy(table, sc_rows, hidden, table.dtype)

    def tc_body(t_ref, o_ref):
        o_ref[...] = t_ref[...]

    tc_out = pl.pallas_call(
        tc_body,
        grid=(tc_rows // _TC_BLK,),
        in_specs=[
            pl.BlockSpec((_TC_BLK, hidden), lambda i: (sc_rows // _TC_BLK + i, 0))
        ],
        out_specs=pl.BlockSpec(
            (_TC_BLK, hidden), lambda i: (sc_rows // _TC_BLK + i, 0)
        ),
        out_shape=jax.ShapeDtypeStruct((seq_length, hidden), table.dtype),
    )(table)

    def stitch_body(s_ref, t_ref, o_ref):
        o_ref[...] = s_ref[...]

    out = pl.pallas_call(
        stitch_body,
        grid=(sc_rows // _TC_BLK,),
        in_specs=[
            pl.BlockSpec((_TC_BLK, hidden), lambda i: (i, 0)),
            pl.BlockSpec(memory_space=pl.ANY),
        ],
        out_specs=pl.BlockSpec((_TC_BLK, hidden), lambda i: (i, 0)),
        out_shape=jax.ShapeDtypeStruct((seq_length, hidden), table.dtype),
        input_output_aliases={1: 0},
    )(sc_out, tc_out)
    return out


# speedup vs baseline: 1.1371x; 1.1371x over previous
import functools

import jax
import jax.numpy as jnp
from jax import lax
from jax.experimental import pallas as pl
from jax.experimental.pallas import tpu as pltpu
from jax.experimental.pallas import tpu_sc as plsc

_CHUNK_ROWS = 32
_NBUF = 2


def kernel(x, table):
    seq_length = x.shape[1]
    num_rows, hidden = table.shape
    seq_length = min(seq_length, num_rows)

    info = plsc.get_sparse_core_info()
    num_workers = info.num_cores * info.num_subcores
    rows_per_w = seq_length // num_workers
    assert rows_per_w * num_workers == seq_length
    assert rows_per_w % (2 * _CHUNK_ROWS) == 0
    n_per_path = rows_per_w // (2 * _CHUNK_ROWS)

    mesh = plsc.VectorSubcoreMesh(core_axis_name="c", subcore_axis_name="s")

    @functools.partial(
        pl.kernel,
        mesh=mesh,
        out_type=jax.ShapeDtypeStruct((seq_length, hidden), table.dtype),
        scratch_types=[
            pltpu.VMEM((_NBUF, _CHUNK_ROWS, hidden), table.dtype),
            pltpu.VMEM_SHARED(
                (info.num_subcores, _NBUF, _CHUNK_ROWS, hidden), table.dtype
            ),
            pltpu.SemaphoreType.DMA((_NBUF,)),
            pltpu.SemaphoreType.DMA((_NBUF,)),
            pltpu.SemaphoreType.DMA((_NBUF,)),
            pltpu.SemaphoreType.DMA((_NBUF,)),
        ],
    )
    def copy_rows(table_hbm, out_hbm, tbuf, shared, tl, ts, sl, ss):
        sid = lax.axis_index("s")
        wid = sid * info.num_cores + lax.axis_index("c")
        base = wid * rows_per_w

        def rows_at(g):
            return pl.ds(base + g * _CHUNK_ROWS, _CHUNK_ROWS)

        def load_a(g, b):
            return pltpu.async_copy(table_hbm.at[rows_at(2 * g)], tbuf.at[b], tl.at[b])

        def store_a(g, b):
            return pltpu.async_copy(tbuf.at[b], out_hbm.at[rows_at(2 * g)], ts.at[b])

        def load_b(g, b):
            return pltpu.async_copy(
                table_hbm.at[rows_at(2 * g + 1)], shared.at[sid, b], sl.at[b]
            )

        def store_b(g, b):
            return pltpu.async_copy(
                shared.at[sid, b], out_hbm.at[rows_at(2 * g + 1)], ss.at[b]
            )

        la = [load_a(g, g) for g in range(min(_NBUF, n_per_path))]
        lb = [load_b(g, g) for g in range(min(_NBUF, n_per_path))]
        sa = [None] * _NBUF
        sb = [None] * _NBUF
        for g in range(n_per_path):
            b = g % _NBUF
            nxt = g + _NBUF
            la[b].wait()
            sa[b] = store_a(g, b)
            lb[b].wait()
            sb[b] = store_b(g, b)
            if nxt < n_per_path:
                sa[b].wait()
                sa[b] = None
                la[b] = load_a(nxt, b)
                sb[b].wait()
                sb[b] = None
                lb[b] = load_b(nxt, b)
        for h in sa + sb:
            if h is not None:
                h.wait()

    return copy_rows(table)
